# baseline (speedup 1.0000x reference)
import functools

import jax
import jax.numpy as jnp
from jax import lax
from jax.experimental import pallas as pl
from jax.experimental.pallas import tpu as pltpu
from jax.experimental.pallas import tpu_sc as plsc

N = 32768
D = 128
K = 1024
NC = 2
NS = 16
NW = NC * NS
ROWS_PER_W = N // NW
CHUNK = 128
NCHUNKS = ROWS_PER_W // CHUNK
GPT = K // NS
HBLK = 2048

_mesh = plsc.VectorSubcoreMesh(
    core_axis_name="c", subcore_axis_name="s", num_cores=NC, num_subcores=NS)

_f32 = jnp.float32


def _fill(ref, rows, width, val):
    v = jnp.full((16,), val, _f32)

    def row(r, _):
        for cc in range(width // 16):
            ref[r, pl.ds(cc * 16, 16)] = v
        return 0

    lax.fori_loop(0, rows, row, 0)


def _hist_body(kb_ref, rc_ref, acc_ref):
    kb = kb_ref[0]
    kbt = jnp.transpose(kb, (1, 0))
    w = (kbt == lax.broadcasted_iota(jnp.int32, (1, K), 1)).astype(_f32)
    ones = jnp.ones((HBLK, 1), _f32)
    part = lax.dot_general(w, ones, (((0,), (0,)), ((), ())),
                           precision=lax.Precision.HIGHEST,
                           preferred_element_type=_f32)

    @pl.when(pl.program_id(0) == 0)
    def _():
        acc_ref[...] = jnp.zeros_like(acc_ref)

    acc_ref[...] += part

    @pl.when(pl.program_id(0) == N // HBLK - 1)
    def _():
        rc = 1.0 / jnp.maximum(acc_ref[...], 1.0)
        rc_ref[...] = jnp.broadcast_to(rc, (K, D))


_hist = pl.pallas_call(
    _hist_body,
    grid=(N // HBLK,),
    in_specs=[pl.BlockSpec((1, 1, HBLK), lambda i: (i, 0, 0))],
    out_specs=pl.BlockSpec((K, D), lambda i: (0, 0)),
    out_shape=jax.ShapeDtypeStruct((K, D), _f32),
    scratch_shapes=[pltpu.VMEM((K, 1), _f32)],
)


@functools.partial(
    pl.kernel,
    out_type=(
        jax.ShapeDtypeStruct((NC, K, D), _f32),
        jax.ShapeDtypeStruct((NC, K, D), _f32),
    ),
    mesh=_mesh,
    scratch_types=(
        pltpu.VMEM((CHUNK,), jnp.int32),
        pltpu.VMEM((CHUNK,), jnp.int32),
        pltpu.VMEM((CHUNK, D), _f32),
        pltpu.VMEM((CHUNK, D), _f32),
        pltpu.VMEM((CHUNK, D), _f32),
        pltpu.VMEM((CHUNK, D), _f32),
        pltpu.VMEM((GPT, D), _f32),
        pltpu.VMEM_SHARED((K, D), _f32),
        pltpu.VMEM_SHARED((K, D), _f32),
        pltpu.SemaphoreType.DMA,
        pltpu.SemaphoreType.DMA,
        pltpu.SemaphoreType.DMA,
        pltpu.SemaphoreType.DMA,
        pltpu.SemaphoreType.DMA,
        pltpu.SemaphoreType.DMA,
        pltpu.SemaphoreType.DMA,
        pltpu.SemaphoreType.DMA,
    ),
)
def _accum(key_hbm, x_hbm, s_out, q_out,
           idx0, idx1, x0, x1, sq0, sq1, zbuf, s_sh, q_sh,
           sem_li0, sem_li1, sem_lx0, sem_lx1, sem_sx0, sem_sx1,
           sem_sq0, sem_sq1):
    c = lax.axis_index("c")
    s = lax.axis_index("s")
    wid = c * NS + s
    base = wid * ROWS_PER_W

    idx = (idx0, idx1)
    xb = (x0, x1)
    sqb = (sq0, sq1)
    sem_li = (sem_li0, sem_li1)
    sem_lx = (sem_lx0, sem_lx1)
    sem_sx = (sem_sx0, sem_sx1)
    sem_sq = (sem_sq0, sem_sq1)

    ld_i = [None] * NCHUNKS
    ld_x = [None] * NCHUNKS
    sc_x = [None] * NCHUNKS
    sc_q = [None] * NCHUNKS

    def issue_load(j):
        p = j % 2
        rb = base + j * CHUNK
        ld_i[j] = pltpu.async_copy(key_hbm.at[pl.ds(rb, CHUNK)], idx[p], sem_li[p])
        ld_x[j] = pltpu.async_copy(x_hbm.at[pl.ds(rb, CHUNK)], xb[p], sem_lx[p])

    issue_load(0)

    _fill(zbuf, GPT, D, 0.0)
    gb = s * GPT
    pltpu.sync_copy(zbuf, s_sh.at[pl.ds(gb, GPT)])
    pltpu.sync_copy(zbuf, q_sh.at[pl.ds(gb, GPT)])
    plsc.subcore_barrier()

    for j in range(NCHUNKS):
        p = j % 2
        ld_i[j].wait()
        ld_x[j].wait()
        sc_x[j] = pltpu.async_copy(xb[p], s_sh.at[idx[p]], sem_sx[p], add=True)
        if j + 1 < NCHUNKS:
            if j >= 1:
                sc_x[j - 1].wait()
                sc_q[j - 1].wait()
            issue_load(j + 1)

        def srow(r, _):
            for cc in range(D // 16):
                sl = pl.ds(cc * 16, 16)
                v = xb[p][r, sl]
                sqb[p][r, sl] = v * v
            return 0

        lax.fori_loop(0, CHUNK, srow, 0)
        sc_q[j] = pltpu.async_copy(sqb[p], q_sh.at[idx[p]], sem_sq[p], add=True)

    for j in (NCHUNKS - 2, NCHUNKS - 1):
        sc_x[j].wait()
        sc_q[j].wait()
    plsc.subcore_barrier()

    pltpu.sync_copy(s_sh.at[pl.ds(gb, GPT)], zbuf)
    pltpu.sync_copy(zbuf, s_out.at[c, pl.ds(gb, GPT)])
    pltpu.sync_copy(q_sh.at[pl.ds(gb, GPT)], zbuf)
    pltpu.sync_copy(zbuf, q_out.at[c, pl.ds(gb, GPT)])


@functools.partial(
    pl.kernel,
    out_type=(
        jax.ShapeDtypeStruct((N, D), _f32),
        jax.ShapeDtypeStruct((N, D), _f32),
    ),
    mesh=_mesh,
    scratch_types=(
        pltpu.VMEM((ROWS_PER_W,), jnp.int32),
        pltpu.VMEM((CHUNK, D), _f32),
        pltpu.VMEM((CHUNK, D), _f32),
        pltpu.VMEM((CHUNK, D), _f32),
        pltpu.VMEM((CHUNK, D), _f32),
        pltpu.VMEM((GPT, D), _f32),
        pltpu.VMEM((GPT, D), _f32),
        pltpu.VMEM((GPT, D), _f32),
        pltpu.VMEM((GPT, D), _f32),
        pltpu.VMEM((GPT, D), _f32),
        pltpu.VMEM_SHARED((K, D), _f32),
        pltpu.VMEM_SHARED((K, D), _f32),
        pltpu.SemaphoreType.DMA,
        pltpu.SemaphoreType.DMA,
        pltpu.SemaphoreType.DMA,
        pltpu.SemaphoreType.DMA,
        pltpu.SemaphoreType.DMA,
        pltpu.SemaphoreType.DMA,
        pltpu.SemaphoreType.DMA,
        pltpu.SemaphoreType.DMA,
        pltpu.SemaphoreType.DMA,
        pltpu.SemaphoreType.DMA,
    ),
)
def _finalize_gather(key_hbm, s2, q2, rc_full, om, ov,
                     idx_all, bm0, bm1, bv0, bv1, t_s, t_t, t_q, t_u,
                     t_r, m_sh, v_sh,
                     sem_li0, sem_li1, sem_gm0, sem_gm1, sem_gv0, sem_gv1,
                     sem_wm0, sem_wm1, sem_wv0, sem_wv1):
    c = lax.axis_index("c")
    s = lax.axis_index("s")
    wid = c * NS + s
    gb = s * GPT
    base = wid * ROWS_PER_W

    bm = (bm0, bm1)
    bv = (bv0, bv1)
    sem_gm = (sem_gm0, sem_gm1)
    sem_gv = (sem_gv0, sem_gv1)
    sem_wm = (sem_wm0, sem_wm1)
    sem_wv = (sem_wv0, sem_wv1)

    ld_i = pltpu.async_copy(
        key_hbm.at[pl.ds(base, ROWS_PER_W)], idx_all, sem_li0)

    ld_s0 = pltpu.async_copy(s2.at[0, pl.ds(gb, GPT)], t_s, sem_gm0)
    ld_s1 = pltpu.async_copy(s2.at[1, pl.ds(gb, GPT)], t_t, sem_gm1)
    ld_q0 = pltpu.async_copy(q2.at[0, pl.ds(gb, GPT)], t_q, sem_gv0)
    ld_q1 = pltpu.async_copy(q2.at[1, pl.ds(gb, GPT)], t_u, sem_gv1)
    ld_rc = pltpu.async_copy(rc_full.at[pl.ds(gb, GPT)], t_r, sem_li1)
    ld_s0.wait()
    ld_s1.wait()
    ld_q0.wait()
    ld_q1.wait()
    ld_rc.wait()

    def frow(r, _):
        for cc in range(D // 16):
            sl = pl.ds(cc * 16, 16)
            rc = t_r[r, sl]
            m = (t_s[r, sl] + t_t[r, sl]) * rc
            t_s[r, sl] = m
            t_q[r, sl] = (t_q[r, sl] + t_u[r, sl]) * rc - m * m
        return 0

    lax.fori_loop(0, GPT, frow, 0)

    pltpu.sync_copy(t_s, m_sh.at[pl.ds(gb, GPT)])
    pltpu.sync_copy(t_q, v_sh.at[pl.ds(gb, GPT)])
    plsc.subcore_barrier()

    ld_i.wait()
    g_m = [None] * NCHUNKS
    g_v = [None] * NCHUNKS
    w_m = [None] * NCHUNKS
    w_v = [None] * NCHUNKS
    for j in range(NCHUNKS):
        p = j % 2
        rb = base + j * CHUNK
        if j >= 2:
            w_m[j - 2].wait()
            w_v[j - 2].wait()
        idx_j = idx_all.at[pl.ds(j * CHUNK, CHUNK)]
        g_m[j] = pltpu.async_copy(m_sh.at[idx_j], bm[p], sem_gm[p])
        g_v[j] = pltpu.async_copy(v_sh.at[idx_j], bv[p], sem_gv[p])
        g_m[j].wait()
        g_v[j].wait()
        w_m[j] = pltpu.async_copy(bm[p], om.at[pl.ds(rb, CHUNK)], sem_wm[p])
        w_v[j] = pltpu.async_copy(bv[p], ov.at[pl.ds(rb, CHUNK)], sem_wv[p])
    for j in (NCHUNKS - 2, NCHUNKS - 1):
        w_m[j].wait()
        w_v[j].wait()


def kernel(group_by_key, stacked_embeddings):
    key = group_by_key.astype(jnp.int32)
    x = stacked_embeddings
    rc_full = _hist(key.reshape(N // HBLK, 1, HBLK))
    s2, q2 = _accum(key, x)
    return _finalize_gather(key, s2, q2, rc_full)


# speedup vs baseline: 2.1865x; 2.1865x over previous
import functools

import jax
import jax.numpy as jnp
from jax import lax
from jax.experimental import pallas as pl
from jax.experimental.pallas import tpu as pltpu
from jax.experimental.pallas import tpu_sc as plsc

N = 32768
D = 128
K = 1024
NC = 2
NS = 16
NW = NC * NS
ROWS_PER_W = N // NW
CHUNK = 128
NCHUNKS = ROWS_PER_W // CHUNK
GPT = K // NS
HBLK = 2048

_mesh = plsc.VectorSubcoreMesh(
    core_axis_name="c", subcore_axis_name="s", num_cores=NC, num_subcores=NS)

_f32 = jnp.float32


def _fill(ref, rows, width, val):
    v = jnp.full((16,), val, _f32)

    def row(r, _):
        for cc in range(width // 16):
            ref[r, pl.ds(cc * 16, 16)] = v
        return 0

    lax.fori_loop(0, rows, row, 0)


def _hist_body(kb_ref, out_ref):
    kb = kb_ref[...]
    hi = kb >> 7
    lo = kb & 127
    u = (hi == lax.broadcasted_iota(jnp.int32, (1, 8), 1)).astype(jnp.bfloat16)
    v = (lo == lax.broadcasted_iota(jnp.int32, (1, 128), 1)).astype(jnp.bfloat16)
    part = lax.dot_general(u, v, (((0,), (0,)), ((), ())),
                           preferred_element_type=_f32)

    @pl.when(pl.program_id(0) == 0)
    def _():
        out_ref[...] = jnp.zeros_like(out_ref)

    out_ref[...] += part


_hist = pl.pallas_call(
    _hist_body,
    grid=(N // HBLK,),
    in_specs=[pl.BlockSpec((HBLK, 1), lambda i: (i, 0))],
    out_specs=pl.BlockSpec((8, 128), lambda i: (0, 0)),
    out_shape=jax.ShapeDtypeStruct((8, 128), _f32),
)


def _rcbc_body(c_ref, rc_ref):
    c = c_ref[...]
    rc = 1.0 / jnp.maximum(c, 1.0)
    rc_ref[...] = jnp.broadcast_to(rc, (K, D))


_rcbc = pl.pallas_call(
    _rcbc_body,
    out_shape=jax.ShapeDtypeStruct((K, D), _f32),
)


@functools.partial(
    pl.kernel,
    out_type=(
        jax.ShapeDtypeStruct((NC, K, D), _f32),
        jax.ShapeDtypeStruct((NC, K, D), _f32),
    ),
    mesh=_mesh,
    scratch_types=(
        pltpu.VMEM((CHUNK,), jnp.int32),
        pltpu.VMEM((CHUNK,), jnp.int32),
        pltpu.VMEM((CHUNK, D), _f32),
        pltpu.VMEM((CHUNK, D), _f32),
        pltpu.VMEM((CHUNK, D), _f32),
        pltpu.VMEM((CHUNK, D), _f32),
        pltpu.VMEM((GPT, D), _f32),
        pltpu.VMEM_SHARED((K, D), _f32),
        pltpu.VMEM_SHARED((K, D), _f32),
        pltpu.SemaphoreType.DMA,
        pltpu.SemaphoreType.DMA,
        pltpu.SemaphoreType.DMA,
        pltpu.SemaphoreType.DMA,
        pltpu.SemaphoreType.DMA,
        pltpu.SemaphoreType.DMA,
        pltpu.SemaphoreType.DMA,
        pltpu.SemaphoreType.DMA,
    ),
)
def _accum(key_hbm, x_hbm, s_out, q_out,
           idx0, idx1, x0, x1, sq0, sq1, zbuf, s_sh, q_sh,
           sem_li0, sem_li1, sem_lx0, sem_lx1, sem_sx0, sem_sx1,
           sem_sq0, sem_sq1):
    c = lax.axis_index("c")
    s = lax.axis_index("s")
    wid = c * NS + s
    base = wid * ROWS_PER_W

    idx = (idx0, idx1)
    xb = (x0, x1)
    sqb = (sq0, sq1)
    sem_li = (sem_li0, sem_li1)
    sem_lx = (sem_lx0, sem_lx1)
    sem_sx = (sem_sx0, sem_sx1)
    sem_sq = (sem_sq0, sem_sq1)

    ld_i = [None] * NCHUNKS
    ld_x = [None] * NCHUNKS
    sc_x = [None] * NCHUNKS
    sc_q = [None] * NCHUNKS

    def issue_load(j):
        p = j % 2
        rb = base + j * CHUNK
        ld_i[j] = pltpu.async_copy(key_hbm.at[pl.ds(rb, CHUNK)], idx[p], sem_li[p])
        ld_x[j] = pltpu.async_copy(x_hbm.at[pl.ds(rb, CHUNK)], xb[p], sem_lx[p])

    issue_load(0)

    _fill(zbuf, GPT, D, 0.0)
    gb = s * GPT
    pltpu.sync_copy(zbuf, s_sh.at[pl.ds(gb, GPT)])
    pltpu.sync_copy(zbuf, q_sh.at[pl.ds(gb, GPT)])
    plsc.subcore_barrier()

    for j in range(NCHUNKS):
        p = j % 2
        ld_i[j].wait()
        ld_x[j].wait()
        sc_x[j] = pltpu.async_copy(xb[p], s_sh.at[idx[p]], sem_sx[p], add=True)
        if j + 1 < NCHUNKS:
            if j >= 1:
                sc_x[j - 1].wait()
                sc_q[j - 1].wait()
            issue_load(j + 1)

        def srow(r, _):
            for cc in range(D // 16):
                sl = pl.ds(cc * 16, 16)
                v = xb[p][r, sl]
                sqb[p][r, sl] = v * v
            return 0

        lax.fori_loop(0, CHUNK, srow, 0)
        sc_q[j] = pltpu.async_copy(sqb[p], q_sh.at[idx[p]], sem_sq[p], add=True)

    for j in (NCHUNKS - 2, NCHUNKS - 1):
        sc_x[j].wait()
        sc_q[j].wait()
    plsc.subcore_barrier()

    pltpu.sync_copy(s_sh.at[pl.ds(gb, GPT)], zbuf)
    pltpu.sync_copy(zbuf, s_out.at[c, pl.ds(gb, GPT)])
    pltpu.sync_copy(q_sh.at[pl.ds(gb, GPT)], zbuf)
    pltpu.sync_copy(zbuf, q_out.at[c, pl.ds(gb, GPT)])


@functools.partial(
    pl.kernel,
    out_type=(
        jax.ShapeDtypeStruct((N, D), _f32),
        jax.ShapeDtypeStruct((N, D), _f32),
    ),
    mesh=_mesh,
    scratch_types=(
        pltpu.VMEM((ROWS_PER_W,), jnp.int32),
        pltpu.VMEM((CHUNK, D), _f32),
        pltpu.VMEM((CHUNK, D), _f32),
        pltpu.VMEM((CHUNK, D), _f32),
        pltpu.VMEM((CHUNK, D), _f32),
        pltpu.VMEM((GPT, D), _f32),
        pltpu.VMEM((GPT, D), _f32),
        pltpu.VMEM((GPT, D), _f32),
        pltpu.VMEM((GPT, D), _f32),
        pltpu.VMEM((GPT, D), _f32),
        pltpu.VMEM_SHARED((K, D), _f32),
        pltpu.VMEM_SHARED((K, D), _f32),
        pltpu.SemaphoreType.DMA,
        pltpu.SemaphoreType.DMA,
        pltpu.SemaphoreType.DMA,
        pltpu.SemaphoreType.DMA,
        pltpu.SemaphoreType.DMA,
        pltpu.SemaphoreType.DMA,
        pltpu.SemaphoreType.DMA,
        pltpu.SemaphoreType.DMA,
        pltpu.SemaphoreType.DMA,
        pltpu.SemaphoreType.DMA,
    ),
)
def _finalize_gather(key_hbm, s2, q2, rc_full, om, ov,
                     idx_all, bm0, bm1, bv0, bv1, t_s, t_t, t_q, t_u,
                     t_r, m_sh, v_sh,
                     sem_li0, sem_li1, sem_gm0, sem_gm1, sem_gv0, sem_gv1,
                     sem_wm0, sem_wm1, sem_wv0, sem_wv1):
    c = lax.axis_index("c")
    s = lax.axis_index("s")
    wid = c * NS + s
    gb = s * GPT
    base = wid * ROWS_PER_W

    bm = (bm0, bm1)
    bv = (bv0, bv1)
    sem_gm = (sem_gm0, sem_gm1)
    sem_gv = (sem_gv0, sem_gv1)
    sem_wm = (sem_wm0, sem_wm1)
    sem_wv = (sem_wv0, sem_wv1)

    ld_i = pltpu.async_copy(
        key_hbm.at[pl.ds(base, ROWS_PER_W)], idx_all, sem_li0)

    ld_s0 = pltpu.async_copy(s2.at[0, pl.ds(gb, GPT)], t_s, sem_gm0)
    ld_s1 = pltpu.async_copy(s2.at[1, pl.ds(gb, GPT)], t_t, sem_gm1)
    ld_q0 = pltpu.async_copy(q2.at[0, pl.ds(gb, GPT)], t_q, sem_gv0)
    ld_q1 = pltpu.async_copy(q2.at[1, pl.ds(gb, GPT)], t_u, sem_gv1)
    ld_rc = pltpu.async_copy(rc_full.at[pl.ds(gb, GPT)], t_r, sem_li1)
    ld_s0.wait()
    ld_s1.wait()
    ld_q0.wait()
    ld_q1.wait()
    ld_rc.wait()

    def frow(r, _):
        for cc in range(D // 16):
            sl = pl.ds(cc * 16, 16)
            rc = t_r[r, sl]
            m = (t_s[r, sl] + t_t[r, sl]) * rc
            t_s[r, sl] = m
            t_q[r, sl] = (t_q[r, sl] + t_u[r, sl]) * rc - m * m
        return 0

    lax.fori_loop(0, GPT, frow, 0)

    pltpu.sync_copy(t_s, m_sh.at[pl.ds(gb, GPT)])
    pltpu.sync_copy(t_q, v_sh.at[pl.ds(gb, GPT)])
    plsc.subcore_barrier()

    ld_i.wait()
    g_m = [None] * NCHUNKS
    g_v = [None] * NCHUNKS
    w_m = [None] * NCHUNKS
    w_v = [None] * NCHUNKS
    for j in range(NCHUNKS):
        p = j % 2
        rb = base + j * CHUNK
        if j >= 2:
            w_m[j - 2].wait()
            w_v[j - 2].wait()
        idx_j = idx_all.at[pl.ds(j * CHUNK, CHUNK)]
        g_m[j] = pltpu.async_copy(m_sh.at[idx_j], bm[p], sem_gm[p])
        g_v[j] = pltpu.async_copy(v_sh.at[idx_j], bv[p], sem_gv[p])
        g_m[j].wait()
        g_v[j].wait()
        w_m[j] = pltpu.async_copy(bm[p], om.at[pl.ds(rb, CHUNK)], sem_wm[p])
        w_v[j] = pltpu.async_copy(bv[p], ov.at[pl.ds(rb, CHUNK)], sem_wv[p])
    for j in (NCHUNKS - 2, NCHUNKS - 1):
        w_m[j].wait()
        w_v[j].wait()


def kernel(group_by_key, stacked_embeddings):
    key = group_by_key.astype(jnp.int32)
    x = stacked_embeddings
    rc_full = _rcbc(_hist(key[:, None]).reshape(K, 1))
    s2, q2 = _accum(key, x)
    return _finalize_gather(key, s2, q2, rc_full)
